# Initial kernel scaffold; baseline (speedup 1.0000x reference)
import jax
import jax.numpy as jnp
from jax.experimental import pallas as pl


def kernel(feat_in, edge_index, W_self, b_self, W_neigh, b_neigh, offset, scale):
    raise NotImplementedError("write your pallas kernel here")


# speedup vs baseline: 5.9769x; 5.9769x over previous
import functools

import jax
import jax.numpy as jnp
from jax import lax
from jax.experimental import pallas as pl
from jax.experimental.pallas import tpu as pltpu
from jax.experimental.pallas import tpu_sc as plsc

N = 10000
D = 128
E = 320000

NC = 2
NS = 16
NW = NC * NS
EPW = E // NW
K = 80
NCHUNK = EPW // K
NP = 10240
RPT = NP // NS
DUMP = 128


def _sc_agg_body(feat_hbm, row_hbm, col_hbm, zf_hbm,
                 psum_hbm, pdeg_hbm,
                 colbuf, rowbuf, featbuf, degbuf, dumpf,
                 accum, sem):
    c = lax.axis_index("c")
    s = lax.axis_index("s")
    wid = s * NC + c
    base = wid * EPW

    pltpu.sync_copy(zf_hbm.at[pl.ds(s * RPT, RPT)], accum.at[pl.ds(s * RPT, RPT)])

    def zbody(r, carry):
        degbuf[pl.ds(r * 16, 16)] = jnp.zeros((16,), jnp.float32)
        return carry
    lax.fori_loop(0, NP // 16, zbody, 0)
    plsc.subcore_barrier()

    ones16 = jnp.ones((16,), jnp.float32)

    def body(ch, carry):
        off = base + ch * K
        pltpu.sync_copy(col_hbm.at[pl.ds(off, K)], colbuf)
        pltpu.sync_copy(row_hbm.at[pl.ds(off, K)], rowbuf)
        pltpu.async_copy(feat_hbm.at[colbuf], featbuf, sem).wait()
        pltpu.sync_copy(featbuf, accum.at[rowbuf], add=True)
        for j in range(K // 16):
            idx = rowbuf[pl.ds(j * 16, 16)]
            plsc.addupdate_scatter(degbuf, [idx], ones16)
        return carry

    lax.fori_loop(0, NCHUNK, body, 0)
    plsc.subcore_barrier()

    out_base = c * NP + s * RPT
    for j in range(RPT // DUMP):
        pltpu.sync_copy(accum.at[pl.ds(s * RPT + j * DUMP, DUMP)], dumpf)
        pltpu.sync_copy(dumpf, psum_hbm.at[pl.ds(out_base + j * DUMP, DUMP)])
    pltpu.sync_copy(degbuf, pdeg_hbm.at[pl.ds(wid * NP, NP)])


_sc_agg = functools.partial(
    pl.kernel,
    out_type=(
        jax.ShapeDtypeStruct((2 * NP, D), jnp.float32),
        jax.ShapeDtypeStruct((NW * NP,), jnp.float32),
    ),
    mesh=plsc.VectorSubcoreMesh(core_axis_name="c", subcore_axis_name="s",
                                num_cores=NC, num_subcores=NS),
    compiler_params=pltpu.CompilerParams(needs_layout_passes=False),
    scratch_types=[
        pltpu.VMEM((K,), jnp.int32),
        pltpu.VMEM((K,), jnp.int32),
        pltpu.VMEM((K, D), jnp.float32),
        pltpu.VMEM((NP,), jnp.float32),
        pltpu.VMEM((DUMP, D), jnp.float32),
        pltpu.VMEM_SHARED((NP, D), jnp.float32),
        pltpu.SemaphoreType.DMA,
    ],
)(_sc_agg_body)


def _tc_dense_body(feat_ref, ps_ref, pd_ref, ws_ref, wn_ref, bs_ref, bn_ref,
                   scs_ref, ofs_ref, scn_ref, ofn_ref, out_ref):
    x = feat_ref[...]
    ns = ps_ref[0] + ps_ref[1]
    dg = jnp.sum(pd_ref[...], axis=1, keepdims=True)
    dg = jnp.maximum(dg, 1.0)
    fn = ns / dg

    dn = (((1,), (1,)), ((), ()))
    hs = lax.dot_general(x, ws_ref[...], dn, preferred_element_type=jnp.float32)
    hs = jnp.maximum(hs + bs_ref[...], 0.0)
    hn = lax.dot_general(fn, wn_ref[...], dn, preferred_element_type=jnp.float32)
    hn = jnp.maximum(hn + bn_ref[...], 0.0)

    def ln(h, sc, of):
        m = jnp.mean(h, axis=1, keepdims=True)
        v = jnp.mean((h - m) ** 2, axis=1, keepdims=True) + 1e-9
        return (h - m) * sc * lax.rsqrt(v) + of

    out_ref[...] = (ln(hs, scs_ref[...], ofs_ref[...])
                    + ln(hn, scn_ref[...], ofn_ref[...]))


BR = 400


def _tc_dense(feat, psum, pdeg, W_self, W_neigh, b_self, b_neigh,
              sc_s, of_s, sc_n, of_n):
    grid = (N // BR,)
    full = lambda shape: pl.BlockSpec(shape, lambda i: (0,) * len(shape))
    return pl.pallas_call(
        _tc_dense_body,
        grid=grid,
        in_specs=[
            pl.BlockSpec((BR, D), lambda i: (i, 0)),
            pl.BlockSpec((2, BR, D), lambda i: (0, i, 0)),
            pl.BlockSpec((BR, NW), lambda i: (i, 0)),
            full((D, D)),
            full((D, D)),
            full((1, D)),
            full((1, D)),
            full((1, D)),
            full((1, D)),
            full((1, D)),
            full((1, D)),
        ],
        out_specs=pl.BlockSpec((BR, D), lambda i: (i, 0)),
        out_shape=jax.ShapeDtypeStruct((N, D), jnp.float32),
    )(feat, psum, pdeg, W_self, W_neigh, b_self, b_neigh,
      sc_s, of_s, sc_n, of_n)


def kernel(feat_in, edge_index, W_self, b_self, W_neigh, b_neigh, offset, scale):
    row = edge_index[0]
    col = edge_index[1]
    zf = jnp.zeros((NP, D), jnp.float32)

    psum, pdeg = _sc_agg(feat_in, row, col, zf)
    psum = psum.reshape(2, NP, D)[:, :N]
    pdeg = pdeg.reshape(NW, NP)[:, :N].T

    return _tc_dense(
        feat_in, psum, pdeg, W_self, W_neigh,
        b_self.reshape(1, D), b_neigh.reshape(1, D),
        scale[:D].reshape(1, D), offset[:D].reshape(1, D),
        scale[D:].reshape(1, D), offset[D:].reshape(1, D),
    )


# speedup vs baseline: 11.7272x; 1.9621x over previous
import functools

import jax
import jax.numpy as jnp
from jax import lax
from jax.experimental import pallas as pl
from jax.experimental.pallas import tpu as pltpu
from jax.experimental.pallas import tpu_sc as plsc

N = 10000
D = 128
E = 320000

NC = 2
NS = 16
NW = NC * NS
EPW = E // NW
K = 80
NCHUNK = EPW // K
G = 5
NGI = NCHUNK // G
R = 2
NP = 10240
RPT = NP // NS


def _sc_agg_body(feat_hbm, row_hbm, col_hbm, zf_hbm,
                 psum_hbm, pdeg_hbm,
                 colbuf, rowbuf, fb, degbuf,
                 accum, semg, semic, semir):
    c = lax.axis_index("c")
    s = lax.axis_index("s")
    wid = s * NC + c

    pltpu.sync_copy(zf_hbm, accum.at[pl.ds(s * RPT, RPT)])

    def zbody(r, carry):
        degbuf[pl.ds(r * 16, 16)] = jnp.zeros((16,), jnp.float32)
        return carry
    lax.fori_loop(0, NP // 16, zbody, 0)
    plsc.subcore_barrier()

    pltpu.sync_copy(col_hbm.at[wid, 0], colbuf.at[pl.ds(0, G)])
    pltpu.sync_copy(row_hbm.at[wid, 0], rowbuf.at[pl.ds(0, G)])
    pltpu.async_copy(col_hbm.at[wid, 1], colbuf.at[pl.ds(G, G)], semic)
    pltpu.async_copy(row_hbm.at[wid, 1], rowbuf.at[pl.ds(G, G)], semir)

    ones16 = jnp.ones((16,), jnp.float32)

    for b in range(R):
        pltpu.async_copy(feat_hbm.at[colbuf.at[b]],
                         fb.at[pl.ds(b * K, K)], semg.at[b])

    def chunk(ch, carry):
        grp = ch // G
        j = ch - grp * G
        slot = lax.rem(grp, 2)
        b = lax.rem(ch, R)

        @pl.when(jnp.logical_and(j == G - R, grp + 1 < NGI))
        def _():
            pltpu.make_async_copy(col_hbm.at[wid, grp + 1],
                                  colbuf.at[pl.ds((1 - slot) * G, G)],
                                  semic).wait()
            pltpu.make_async_copy(row_hbm.at[wid, grp + 1],
                                  rowbuf.at[pl.ds((1 - slot) * G, G)],
                                  semir).wait()

        rw = slot * G + j
        pltpu.make_async_copy(feat_hbm.at[colbuf.at[rw]],
                              fb.at[pl.ds(b * K, K)], semg.at[b]).wait()
        pltpu.sync_copy(fb.at[pl.ds(b * K, K)],
                        accum.at[rowbuf.at[rw]], add=True)
        for t in range(K // 16):
            idx = rowbuf[rw, pl.ds(t * 16, 16)]
            plsc.addupdate_scatter(degbuf, [idx], ones16)

        nxt = ch + R

        @pl.when(nxt < NCHUNK)
        def _():
            gn = nxt // G
            jn = nxt - gn * G
            sn = lax.rem(gn, 2)
            pltpu.async_copy(feat_hbm.at[colbuf.at[sn * G + jn]],
                             fb.at[pl.ds(b * K, K)], semg.at[b])

        @pl.when(jnp.logical_and(j == G - 1, grp + 2 < NGI))
        def _():
            pltpu.async_copy(col_hbm.at[wid, grp + 2],
                             colbuf.at[pl.ds(slot * G, G)], semic)
            pltpu.async_copy(row_hbm.at[wid, grp + 2],
                             rowbuf.at[pl.ds(slot * G, G)], semir)

        return carry

    lax.fori_loop(0, NCHUNK, chunk, 0)
    plsc.subcore_barrier()

    out_base = c * NP + s * RPT
    pltpu.sync_copy(accum.at[pl.ds(s * RPT, RPT)],
                    psum_hbm.at[pl.ds(out_base, RPT)])
    pltpu.sync_copy(degbuf, pdeg_hbm.at[pl.ds(wid * NP, NP)])


_sc_agg = functools.partial(
    pl.kernel,
    out_type=(
        jax.ShapeDtypeStruct((2 * NP, D), jnp.float32),
        jax.ShapeDtypeStruct((NW * NP,), jnp.float32),
    ),
    mesh=plsc.VectorSubcoreMesh(core_axis_name="c", subcore_axis_name="s",
                                num_cores=NC, num_subcores=NS),
    compiler_params=pltpu.CompilerParams(needs_layout_passes=False),
    scratch_types=[
        pltpu.VMEM((2 * G, K), jnp.int32),
        pltpu.VMEM((2 * G, K), jnp.int32),
        pltpu.VMEM((R * K, D), jnp.float32),
        pltpu.VMEM((NP,), jnp.float32),
        pltpu.VMEM_SHARED((NP, D), jnp.float32),
        pltpu.SemaphoreType.DMA((R,)),
        pltpu.SemaphoreType.DMA,
        pltpu.SemaphoreType.DMA,
    ],
)(_sc_agg_body)


def _tc_dense_body(feat_ref, ps_ref, pd_ref, ws_ref, wn_ref, bs_ref, bn_ref,
                   scs_ref, ofs_ref, scn_ref, ofn_ref, out_ref):
    x = feat_ref[...]
    ns = ps_ref[0] + ps_ref[1]
    dg = jnp.sum(pd_ref[...], axis=1, keepdims=True)
    dg = jnp.maximum(dg, 1.0)
    fn = ns / dg

    dn = (((1,), (1,)), ((), ()))
    hs = lax.dot_general(x, ws_ref[...], dn, preferred_element_type=jnp.float32)
    hs = jnp.maximum(hs + bs_ref[...], 0.0)
    hn = lax.dot_general(fn, wn_ref[...], dn, preferred_element_type=jnp.float32)
    hn = jnp.maximum(hn + bn_ref[...], 0.0)

    def ln(h, sc, of):
        m = jnp.mean(h, axis=1, keepdims=True)
        v = jnp.mean((h - m) ** 2, axis=1, keepdims=True) + 1e-9
        return (h - m) * sc * lax.rsqrt(v) + of

    out_ref[...] = (ln(hs, scs_ref[...], ofs_ref[...])
                    + ln(hn, scn_ref[...], ofn_ref[...]))


BR = 400


def _tc_dense(feat, psum, pdeg, W_self, W_neigh, b_self, b_neigh,
              sc_s, of_s, sc_n, of_n):
    grid = (N // BR,)
    full = lambda shape: pl.BlockSpec(shape, lambda i: (0,) * len(shape))
    return pl.pallas_call(
        _tc_dense_body,
        grid=grid,
        in_specs=[
            pl.BlockSpec((BR, D), lambda i: (i, 0)),
            pl.BlockSpec((2, BR, D), lambda i: (0, i, 0)),
            pl.BlockSpec((BR, NW), lambda i: (i, 0)),
            full((D, D)),
            full((D, D)),
            full((1, D)),
            full((1, D)),
            full((1, D)),
            full((1, D)),
            full((1, D)),
            full((1, D)),
        ],
        out_specs=pl.BlockSpec((BR, D), lambda i: (i, 0)),
        out_shape=jax.ShapeDtypeStruct((N, D), jnp.float32),
    )(feat, psum, pdeg, W_self, W_neigh, b_self, b_neigh,
      sc_s, of_s, sc_n, of_n)


def kernel(feat_in, edge_index, W_self, b_self, W_neigh, b_neigh, offset, scale):
    row = edge_index[0].reshape(NW, NGI, G, K)
    col = edge_index[1].reshape(NW, NGI, G, K)
    zf = jnp.zeros((RPT, D), jnp.float32)

    psum, pdeg = _sc_agg(feat_in, row, col, zf)
    psum = psum.reshape(2, NP, D)[:, :N]
    pdeg = pdeg.reshape(NW, NP)[:, :N].T

    return _tc_dense(
        feat_in, psum, pdeg, W_self, W_neigh,
        b_self.reshape(1, D), b_neigh.reshape(1, D),
        scale[:D].reshape(1, D), offset[:D].reshape(1, D),
        scale[D:].reshape(1, D), offset[D:].reshape(1, D),
    )


# speedup vs baseline: 13.0055x; 1.1090x over previous
import functools

import jax
import jax.numpy as jnp
from jax import lax
from jax.experimental import pallas as pl
from jax.experimental.pallas import tpu as pltpu
from jax.experimental.pallas import tpu_sc as plsc

N = 10000
D = 128
E = 320000

NC = 2
NS = 16
NW = NC * NS
EPW = E // NW
K = 80
NCHUNK = EPW // K
G = 5
NGI = NCHUNK // G
P = 2
Q = 3
NP = 10240
RPT = NP // NS


def _sc_agg_body(feat_hbm, row_hbm, col_hbm, zf_hbm, zd_hbm,
                 psum_hbm, pdeg_hbm,
                 colbuf, rowbuf, fb, degbuf,
                 accum, semg, semsc, semic, semir):
    c = lax.axis_index("c")
    s = lax.axis_index("s")
    wid = s * NC + c

    pltpu.sync_copy(zf_hbm, accum.at[pl.ds(s * RPT, RPT)])
    pltpu.sync_copy(zd_hbm, degbuf)
    plsc.subcore_barrier()

    pltpu.sync_copy(col_hbm.at[wid, 0], colbuf.at[pl.ds(0, G)])
    pltpu.sync_copy(row_hbm.at[wid, 0], rowbuf.at[pl.ds(0, G)])
    pltpu.async_copy(col_hbm.at[wid, 1], colbuf.at[pl.ds(G, G)], semic)
    pltpu.async_copy(row_hbm.at[wid, 1], rowbuf.at[pl.ds(G, G)], semir)

    ones16 = jnp.ones((16,), jnp.float32)

    for b in range(P):
        pltpu.async_copy(feat_hbm.at[colbuf.at[b]],
                         fb.at[pl.ds(b * K, K)], semg.at[b])

    def chunk(ch, carry):
        grp = ch // G
        j = ch - grp * G
        slot = lax.rem(grp, 2)
        sg = lax.rem(ch, Q)

        @pl.when(jnp.logical_and(j == G - P, grp + 1 < NGI))
        def _():
            pltpu.make_async_copy(col_hbm.at[wid, grp + 1],
                                  colbuf.at[pl.ds((1 - slot) * G, G)],
                                  semic).wait()
            pltpu.make_async_copy(row_hbm.at[wid, grp + 1],
                                  rowbuf.at[pl.ds((1 - slot) * G, G)],
                                  semir).wait()

        rw = slot * G + j
        pltpu.make_async_copy(feat_hbm.at[colbuf.at[rw]],
                              fb.at[pl.ds(sg * K, K)], semg.at[sg]).wait()
        pltpu.async_copy(fb.at[pl.ds(sg * K, K)],
                         accum.at[rowbuf.at[rw]], semsc.at[sg], add=True)
        for t in range(K // 16):
            idx = rowbuf[rw, pl.ds(t * 16, 16)]
            plsc.addupdate_scatter(degbuf, [idx], ones16)

        nxt = ch + P
        sn_fb = lax.rem(nxt, Q)

        @pl.when(jnp.logical_and(nxt < NCHUNK, ch >= Q - P))
        def _():
            pltpu.make_async_copy(fb.at[pl.ds(sn_fb * K, K)],
                                  accum.at[rowbuf.at[rw]],
                                  semsc.at[sn_fb]).wait()

        @pl.when(nxt < NCHUNK)
        def _():
            gn = nxt // G
            jn = nxt - gn * G
            sn = lax.rem(gn, 2)
            pltpu.async_copy(feat_hbm.at[colbuf.at[sn * G + jn]],
                             fb.at[pl.ds(sn_fb * K, K)], semg.at[sn_fb])

        @pl.when(jnp.logical_and(j == G - 1, grp + 2 < NGI))
        def _():
            pltpu.async_copy(col_hbm.at[wid, grp + 2],
                             colbuf.at[pl.ds(slot * G, G)], semic)
            pltpu.async_copy(row_hbm.at[wid, grp + 2],
                             rowbuf.at[pl.ds(slot * G, G)], semir)

        return carry

    lax.fori_loop(0, NCHUNK, chunk, 0)

    for q in range(Q):
        pltpu.make_async_copy(fb.at[pl.ds(q * K, K)],
                              accum.at[rowbuf.at[0]], semsc.at[q]).wait()
    plsc.subcore_barrier()

    out_base = c * NP + s * RPT
    pltpu.sync_copy(accum.at[pl.ds(s * RPT, RPT)],
                    psum_hbm.at[pl.ds(out_base, RPT)])
    pltpu.sync_copy(degbuf, pdeg_hbm.at[pl.ds(wid * NP, NP)])


_sc_agg = functools.partial(
    pl.kernel,
    out_type=(
        jax.ShapeDtypeStruct((2 * NP, D), jnp.float32),
        jax.ShapeDtypeStruct((NW * NP,), jnp.float32),
    ),
    mesh=plsc.VectorSubcoreMesh(core_axis_name="c", subcore_axis_name="s",
                                num_cores=NC, num_subcores=NS),
    compiler_params=pltpu.CompilerParams(needs_layout_passes=False),
    scratch_types=[
        pltpu.VMEM((2 * G, K), jnp.int32),
        pltpu.VMEM((2 * G, K), jnp.int32),
        pltpu.VMEM((Q * K, D), jnp.float32),
        pltpu.VMEM((NP,), jnp.float32),
        pltpu.VMEM_SHARED((NP, D), jnp.float32),
        pltpu.SemaphoreType.DMA((Q,)),
        pltpu.SemaphoreType.DMA((Q,)),
        pltpu.SemaphoreType.DMA,
        pltpu.SemaphoreType.DMA,
    ],
)(_sc_agg_body)


def _tc_dense_body(feat_ref, ps_ref, pd_ref, ws_ref, wn_ref, bs_ref, bn_ref,
                   scs_ref, ofs_ref, scn_ref, ofn_ref, out_ref):
    x = feat_ref[...]
    ns = ps_ref[0] + ps_ref[1]
    dg = jnp.sum(pd_ref[...], axis=1, keepdims=True)
    dg = jnp.maximum(dg, 1.0)
    fn = ns / dg

    dn = (((1,), (1,)), ((), ()))
    hs = lax.dot_general(x, ws_ref[...], dn, preferred_element_type=jnp.float32)
    hs = jnp.maximum(hs + bs_ref[...], 0.0)
    hn = lax.dot_general(fn, wn_ref[...], dn, preferred_element_type=jnp.float32)
    hn = jnp.maximum(hn + bn_ref[...], 0.0)

    def ln(h, sc, of):
        m = jnp.mean(h, axis=1, keepdims=True)
        v = jnp.mean((h - m) ** 2, axis=1, keepdims=True) + 1e-9
        return (h - m) * sc * lax.rsqrt(v) + of

    out_ref[...] = (ln(hs, scs_ref[...], ofs_ref[...])
                    + ln(hn, scn_ref[...], ofn_ref[...]))


BR = 400


def _tc_dense(feat, psum, pdeg, W_self, W_neigh, b_self, b_neigh,
              sc_s, of_s, sc_n, of_n):
    grid = (N // BR,)
    full = lambda shape: pl.BlockSpec(shape, lambda i: (0,) * len(shape))
    return pl.pallas_call(
        _tc_dense_body,
        grid=grid,
        in_specs=[
            pl.BlockSpec((BR, D), lambda i: (i, 0)),
            pl.BlockSpec((2, BR, D), lambda i: (0, i, 0)),
            pl.BlockSpec((BR, NW), lambda i: (i, 0)),
            full((D, D)),
            full((D, D)),
            full((1, D)),
            full((1, D)),
            full((1, D)),
            full((1, D)),
            full((1, D)),
            full((1, D)),
        ],
        out_specs=pl.BlockSpec((BR, D), lambda i: (i, 0)),
        out_shape=jax.ShapeDtypeStruct((N, D), jnp.float32),
    )(feat, psum, pdeg, W_self, W_neigh, b_self, b_neigh,
      sc_s, of_s, sc_n, of_n)


def kernel(feat_in, edge_index, W_self, b_self, W_neigh, b_neigh, offset, scale):
    row = edge_index[0].reshape(NW, NGI, G, K)
    col = edge_index[1].reshape(NW, NGI, G, K)
    zf = jnp.zeros((RPT, D), jnp.float32)
    zd = jnp.zeros((NP,), jnp.float32)

    psum, pdeg = _sc_agg(feat_in, row, col, zf, zd)
    psum = psum.reshape(2, NP, D)[:, :N]
    pdeg = pdeg.reshape(NW, NP)[:, :N].T

    return _tc_dense(
        feat_in, psum, pdeg, W_self, W_neigh,
        b_self.reshape(1, D), b_neigh.reshape(1, D),
        scale[:D].reshape(1, D), offset[:D].reshape(1, D),
        scale[D:].reshape(1, D), offset[D:].reshape(1, D),
    )


# speedup vs baseline: 13.3135x; 1.0237x over previous
import functools

import jax
import jax.numpy as jnp
from jax import lax
from jax.experimental import pallas as pl
from jax.experimental.pallas import tpu as pltpu
from jax.experimental.pallas import tpu_sc as plsc

N = 10000
D = 128
E = 320000

NC = 2
NS = 16
NW = NC * NS
EPW = E // NW
K = 80
NCHUNK = EPW // K
G = 5
NGI = NCHUNK // G
P = 2
Q = 3
NP = 10240
RPT = NP // NS


def _sc_agg_body(feat_hbm, row_hbm, col_hbm, zf_hbm, zd_hbm,
                 psum_hbm, pdeg_hbm,
                 colbuf, rowbuf, fb, degbuf,
                 accum, semg, semsc, semic, semir):
    c = lax.axis_index("c")
    s = lax.axis_index("s")
    wid = s * NC + c

    pltpu.sync_copy(zf_hbm, accum.at[pl.ds(s * RPT, RPT)])
    pltpu.sync_copy(zd_hbm, degbuf)
    plsc.subcore_barrier()

    pltpu.sync_copy(col_hbm.at[wid, 0], colbuf.at[pl.ds(0, G)])
    pltpu.sync_copy(row_hbm.at[wid, 0], rowbuf.at[pl.ds(0, G)])
    pltpu.async_copy(col_hbm.at[wid, 1], colbuf.at[pl.ds(G, G)], semic)
    pltpu.async_copy(row_hbm.at[wid, 1], rowbuf.at[pl.ds(G, G)], semir)

    ones16 = jnp.ones((16,), jnp.float32)

    for b in range(P):
        pltpu.async_copy(feat_hbm.at[colbuf.at[b]],
                         fb.at[pl.ds(b * K, K)], semg.at[b])

    def chunk(ch, carry):
        grp = ch // G
        j = ch - grp * G
        slot = lax.rem(grp, 2)
        sg = lax.rem(ch, Q)

        @pl.when(jnp.logical_and(j == G - P, grp + 1 < NGI))
        def _():
            pltpu.make_async_copy(col_hbm.at[wid, grp + 1],
                                  colbuf.at[pl.ds((1 - slot) * G, G)],
                                  semic).wait()
            pltpu.make_async_copy(row_hbm.at[wid, grp + 1],
                                  rowbuf.at[pl.ds((1 - slot) * G, G)],
                                  semir).wait()

        rw = slot * G + j
        pltpu.make_async_copy(feat_hbm.at[colbuf.at[rw]],
                              fb.at[pl.ds(sg * K, K)], semg.at[sg]).wait()
        pltpu.async_copy(fb.at[pl.ds(sg * K, K)],
                         accum.at[rowbuf.at[rw]], semsc.at[sg], add=True)
        for t in range(K // 16):
            idx = rowbuf[rw, pl.ds(t * 16, 16)]
            plsc.addupdate_scatter(degbuf, [idx], ones16)

        nxt = ch + P
        sn_fb = lax.rem(nxt, Q)

        @pl.when(jnp.logical_and(nxt < NCHUNK, ch >= Q - P))
        def _():
            pltpu.make_async_copy(fb.at[pl.ds(sn_fb * K, K)],
                                  accum.at[rowbuf.at[rw]],
                                  semsc.at[sn_fb]).wait()

        @pl.when(nxt < NCHUNK)
        def _():
            gn = nxt // G
            jn = nxt - gn * G
            sn = lax.rem(gn, 2)
            pltpu.async_copy(feat_hbm.at[colbuf.at[sn * G + jn]],
                             fb.at[pl.ds(sn_fb * K, K)], semg.at[sn_fb])

        @pl.when(jnp.logical_and(j == G - 1, grp + 2 < NGI))
        def _():
            pltpu.async_copy(col_hbm.at[wid, grp + 2],
                             colbuf.at[pl.ds(slot * G, G)], semic)
            pltpu.async_copy(row_hbm.at[wid, grp + 2],
                             rowbuf.at[pl.ds(slot * G, G)], semir)

        return carry

    lax.fori_loop(0, NCHUNK, chunk, 0)

    for q in range(Q):
        pltpu.make_async_copy(fb.at[pl.ds(q * K, K)],
                              accum.at[rowbuf.at[0]], semsc.at[q]).wait()
    plsc.subcore_barrier()

    out_base = c * NP + s * RPT
    pltpu.sync_copy(accum.at[pl.ds(s * RPT, RPT)],
                    psum_hbm.at[pl.ds(out_base, RPT)])
    pltpu.sync_copy(degbuf, pdeg_hbm.at[pl.ds(wid * NP, NP)])


_sc_agg = functools.partial(
    pl.kernel,
    out_type=(
        jax.ShapeDtypeStruct((2 * NP, D), jnp.float32),
        jax.ShapeDtypeStruct((NW * NP,), jnp.float32),
    ),
    mesh=plsc.VectorSubcoreMesh(core_axis_name="c", subcore_axis_name="s",
                                num_cores=NC, num_subcores=NS),
    compiler_params=pltpu.CompilerParams(needs_layout_passes=False),
    scratch_types=[
        pltpu.VMEM((2 * G, K), jnp.int32),
        pltpu.VMEM((2 * G, K), jnp.int32),
        pltpu.VMEM((Q * K, D), jnp.float32),
        pltpu.VMEM((NP,), jnp.float32),
        pltpu.VMEM_SHARED((NP, D), jnp.float32),
        pltpu.SemaphoreType.DMA((Q,)),
        pltpu.SemaphoreType.DMA((Q,)),
        pltpu.SemaphoreType.DMA,
        pltpu.SemaphoreType.DMA,
    ],
)(_sc_agg_body)


def _tc_deg_body(pd_ref, ones_ref, dg_ref):
    dn0 = (((0,), (0,)), ((), ()))
    dg = lax.dot_general(pd_ref[...], ones_ref[...], dn0,
                         preferred_element_type=jnp.float32)
    dg_ref[...] = jnp.maximum(dg, 1.0)


def _tc_deg(pdeg):
    return pl.pallas_call(
        _tc_deg_body,
        out_shape=jax.ShapeDtypeStruct((NP, 1), jnp.float32),
    )(pdeg, jnp.ones((NW, 1), jnp.float32))


def _tc_dense_body(feat_ref, ps_ref, dg_ref, ws_ref, wn_ref,
                   bs_ref, bn_ref, scs_ref, ofs_ref, scn_ref, ofn_ref,
                   out_ref):
    x = feat_ref[...]
    ns = ps_ref[0] + ps_ref[1]
    fn = ns / dg_ref[...]

    dn = (((1,), (1,)), ((), ()))
    hs = lax.dot_general(x, ws_ref[...], dn, preferred_element_type=jnp.float32)
    hs = jnp.maximum(hs + bs_ref[...], 0.0)
    hn = lax.dot_general(fn, wn_ref[...], dn, preferred_element_type=jnp.float32)
    hn = jnp.maximum(hn + bn_ref[...], 0.0)

    def ln(h, sc, of):
        m = jnp.mean(h, axis=1, keepdims=True)
        v = jnp.mean((h - m) ** 2, axis=1, keepdims=True) + 1e-9
        return (h - m) * sc * lax.rsqrt(v) + of

    out_ref[...] = (ln(hs, scs_ref[...], ofs_ref[...])
                    + ln(hn, scn_ref[...], ofn_ref[...]))


BR = 400


def _tc_dense(feat, psum, deg, W_self, W_neigh, b_self, b_neigh,
              sc_s, of_s, sc_n, of_n):
    grid = (N // BR,)
    full = lambda shape: pl.BlockSpec(shape, lambda i: (0,) * len(shape))
    return pl.pallas_call(
        _tc_dense_body,
        grid=grid,
        in_specs=[
            pl.BlockSpec((BR, D), lambda i: (i, 0)),
            pl.BlockSpec((2, BR, D), lambda i: (0, i, 0)),
            pl.BlockSpec((BR, 1), lambda i: (i, 0)),
            full((D, D)),
            full((D, D)),
            full((1, D)),
            full((1, D)),
            full((1, D)),
            full((1, D)),
            full((1, D)),
            full((1, D)),
        ],
        out_specs=pl.BlockSpec((BR, D), lambda i: (i, 0)),
        out_shape=jax.ShapeDtypeStruct((N, D), jnp.float32),
    )(feat, psum, deg, W_self, W_neigh, b_self, b_neigh,
      sc_s, of_s, sc_n, of_n)


def kernel(feat_in, edge_index, W_self, b_self, W_neigh, b_neigh, offset, scale):
    row = edge_index[0].reshape(NW, NGI, G, K)
    col = edge_index[1].reshape(NW, NGI, G, K)
    zf = jnp.zeros((RPT, D), jnp.float32)
    zd = jnp.zeros((NP,), jnp.float32)

    psum, pdeg = _sc_agg(feat_in, row, col, zf, zd)
    psum = psum.reshape(2, NP, D)
    deg = _tc_deg(pdeg.reshape(NW, NP))

    return _tc_dense(
        feat_in, psum, deg, W_self, W_neigh,
        b_self.reshape(1, D), b_neigh.reshape(1, D),
        scale[:D].reshape(1, D), offset[:D].reshape(1, D),
        scale[D:].reshape(1, D), offset[D:].reshape(1, D),
    )


# speedup vs baseline: 13.5978x; 1.0214x over previous
import functools

import jax
import jax.numpy as jnp
from jax import lax
from jax.experimental import pallas as pl
from jax.experimental.pallas import tpu as pltpu
from jax.experimental.pallas import tpu_sc as plsc

N = 10000
D = 128
E = 320000

NC = 2
NS = 16
NW = NC * NS
EPW = E // NW
K = 80
NCHUNK = EPW // K
G = 5
NGI = NCHUNK // G
P = 2
Q = 3
NP = 10240
RPT = NP // NS


def _sc_agg_body(feat_hbm, row_hbm, col_hbm, zf_hbm, zd_hbm,
                 psum_hbm, pdeg_hbm,
                 colbuf, rowbuf, fb, degbuf,
                 accum, semg, semsc, semic, semir):
    c = lax.axis_index("c")
    s = lax.axis_index("s")
    wid = s * NC + c

    pltpu.sync_copy(zf_hbm, accum.at[pl.ds(s * RPT, RPT)])
    pltpu.sync_copy(zd_hbm, degbuf)
    plsc.subcore_barrier()

    pltpu.sync_copy(col_hbm.at[wid, 0], colbuf.at[pl.ds(0, G)])
    pltpu.sync_copy(row_hbm.at[wid, 0], rowbuf.at[pl.ds(0, G)])
    pltpu.async_copy(col_hbm.at[wid, 1], colbuf.at[pl.ds(G, G)], semic)
    pltpu.async_copy(row_hbm.at[wid, 1], rowbuf.at[pl.ds(G, G)], semir)

    ones16 = jnp.ones((16,), jnp.float32)

    for b in range(P):
        pltpu.async_copy(feat_hbm.at[colbuf.at[b]],
                         fb.at[pl.ds(b * K, K)], semg.at[b])

    def chunk(ch, carry):
        grp = ch // G
        j = ch - grp * G
        slot = lax.rem(grp, 2)
        sg = lax.rem(ch, Q)

        @pl.when(jnp.logical_and(j == G - P, grp + 1 < NGI))
        def _():
            pltpu.make_async_copy(col_hbm.at[wid, grp + 1],
                                  colbuf.at[pl.ds((1 - slot) * G, G)],
                                  semic).wait()
            pltpu.make_async_copy(row_hbm.at[wid, grp + 1],
                                  rowbuf.at[pl.ds((1 - slot) * G, G)],
                                  semir).wait()

        rw = slot * G + j
        pltpu.make_async_copy(feat_hbm.at[colbuf.at[rw]],
                              fb.at[pl.ds(sg * K, K)], semg.at[sg]).wait()
        pltpu.async_copy(fb.at[pl.ds(sg * K, K)],
                         accum.at[rowbuf.at[rw]], semsc.at[sg], add=True)
        for t in range(K // 16):
            idx = rowbuf[rw, pl.ds(t * 16, 16)]
            plsc.addupdate_scatter(degbuf, [idx], ones16)

        nxt = ch + P
        sn_fb = lax.rem(nxt, Q)

        @pl.when(jnp.logical_and(nxt < NCHUNK, ch >= Q - P))
        def _():
            pltpu.make_async_copy(fb.at[pl.ds(sn_fb * K, K)],
                                  accum.at[rowbuf.at[rw]],
                                  semsc.at[sn_fb]).wait()

        @pl.when(nxt < NCHUNK)
        def _():
            gn = nxt // G
            jn = nxt - gn * G
            sn = lax.rem(gn, 2)
            pltpu.async_copy(feat_hbm.at[colbuf.at[sn * G + jn]],
                             fb.at[pl.ds(sn_fb * K, K)], semg.at[sn_fb])

        @pl.when(jnp.logical_and(j == G - 1, grp + 2 < NGI))
        def _():
            pltpu.async_copy(col_hbm.at[wid, grp + 2],
                             colbuf.at[pl.ds(slot * G, G)], semic)
            pltpu.async_copy(row_hbm.at[wid, grp + 2],
                             rowbuf.at[pl.ds(slot * G, G)], semir)

        return carry

    lax.fori_loop(0, NCHUNK, chunk, 0)

    for q in range(Q):
        pltpu.make_async_copy(fb.at[pl.ds(q * K, K)],
                              accum.at[rowbuf.at[0]], semsc.at[q]).wait()
    plsc.subcore_barrier()

    out_base = c * NP + s * RPT
    pltpu.sync_copy(accum.at[pl.ds(s * RPT, RPT)],
                    psum_hbm.at[pl.ds(out_base, RPT)])
    pltpu.sync_copy(degbuf, pdeg_hbm.at[pl.ds(wid * NP, NP)])


_sc_agg = functools.partial(
    pl.kernel,
    out_type=(
        jax.ShapeDtypeStruct((2 * NP, D), jnp.float32),
        jax.ShapeDtypeStruct((NW * NP,), jnp.float32),
    ),
    mesh=plsc.VectorSubcoreMesh(core_axis_name="c", subcore_axis_name="s",
                                num_cores=NC, num_subcores=NS),
    compiler_params=pltpu.CompilerParams(needs_layout_passes=False),
    scratch_types=[
        pltpu.VMEM((2 * G, K), jnp.int32),
        pltpu.VMEM((2 * G, K), jnp.int32),
        pltpu.VMEM((Q * K, D), jnp.float32),
        pltpu.VMEM((NP,), jnp.float32),
        pltpu.VMEM_SHARED((NP, D), jnp.float32),
        pltpu.SemaphoreType.DMA((Q,)),
        pltpu.SemaphoreType.DMA((Q,)),
        pltpu.SemaphoreType.DMA,
        pltpu.SemaphoreType.DMA,
    ],
)(_sc_agg_body)


def _ln(h, sc, of):
    m = jnp.mean(h, axis=1, keepdims=True)
    v = jnp.mean((h - m) ** 2, axis=1, keepdims=True) + 1e-9
    return (h - m) * sc * lax.rsqrt(v) + of


_DN = (((1,), (1,)), ((), ()))


def _tc_self_body(feat_ref, ws_ref, bs_ref, scs_ref, ofs_ref, out_ref):
    hs = lax.dot_general(feat_ref[...], ws_ref[...], _DN,
                         preferred_element_type=jnp.float32)
    hs = jnp.maximum(hs + bs_ref[...], 0.0)
    out_ref[...] = _ln(hs, scs_ref[...], ofs_ref[...])


def _tc_neigh_body(ps_ref, pd_ref, hs_ref, wn_ref, bn_ref, scn_ref, ofn_ref,
                   out_ref, dg_ref):
    i = pl.program_id(0)

    @pl.when(i == 0)
    def _():
        dn0 = (((0,), (0,)), ((), ()))
        dg = lax.dot_general(pd_ref[...], jnp.ones((NW, 1), jnp.float32),
                             dn0, preferred_element_type=jnp.float32)
        dg_ref[...] = jnp.maximum(dg, 1.0)

    ns = ps_ref[0] + ps_ref[1]
    fn = ns / dg_ref[pl.ds(i * BR, BR), :]
    hn = lax.dot_general(fn, wn_ref[...], _DN,
                         preferred_element_type=jnp.float32)
    hn = jnp.maximum(hn + bn_ref[...], 0.0)
    out_ref[...] = hs_ref[...] + _ln(hn, scn_ref[...], ofn_ref[...])


BR = 400


def _tc_self(feat, W_self, b_self, sc_s, of_s):
    full = lambda shape: pl.BlockSpec(shape, lambda i: (0,) * len(shape))
    return pl.pallas_call(
        _tc_self_body,
        grid=(N // BR,),
        in_specs=[
            pl.BlockSpec((BR, D), lambda i: (i, 0)),
            full((D, D)),
            full((1, D)),
            full((1, D)),
            full((1, D)),
        ],
        out_specs=pl.BlockSpec((BR, D), lambda i: (i, 0)),
        out_shape=jax.ShapeDtypeStruct((N, D), jnp.float32),
    )(feat, W_self, b_self, sc_s, of_s)


def _tc_neigh(psum, pdeg, hself, W_neigh, b_neigh, sc_n, of_n):
    full = lambda shape: pl.BlockSpec(shape, lambda i: (0,) * len(shape))
    return pl.pallas_call(
        _tc_neigh_body,
        grid=(N // BR,),
        in_specs=[
            pl.BlockSpec((2, BR, D), lambda i: (0, i, 0)),
            full((NW, NP)),
            pl.BlockSpec((BR, D), lambda i: (i, 0)),
            full((D, D)),
            full((1, D)),
            full((1, D)),
            full((1, D)),
        ],
        out_specs=pl.BlockSpec((BR, D), lambda i: (i, 0)),
        out_shape=jax.ShapeDtypeStruct((N, D), jnp.float32),
        scratch_shapes=[pltpu.VMEM((NP, 1), jnp.float32)],
    )(psum, pdeg, hself, W_neigh, b_neigh, sc_n, of_n)


def kernel(feat_in, edge_index, W_self, b_self, W_neigh, b_neigh, offset, scale):
    row = edge_index[0].reshape(NW, NGI, G, K)
    col = edge_index[1].reshape(NW, NGI, G, K)
    zf = jnp.zeros((RPT, D), jnp.float32)
    zd = jnp.zeros((NP,), jnp.float32)

    hself = _tc_self(feat_in, W_self, b_self.reshape(1, D),
                     scale[:D].reshape(1, D), offset[:D].reshape(1, D))
    psum, pdeg = _sc_agg(feat_in, row, col, zf, zd)

    return _tc_neigh(
        psum.reshape(2, NP, D), pdeg.reshape(NW, NP), hself,
        W_neigh, b_neigh.reshape(1, D),
        scale[D:].reshape(1, D), offset[D:].reshape(1, D),
    )


# speedup vs baseline: 14.7476x; 1.0846x over previous
import functools

import jax
import jax.numpy as jnp
from jax import lax
from jax.experimental import pallas as pl
from jax.experimental.pallas import tpu as pltpu
from jax.experimental.pallas import tpu_sc as plsc

N = 10000
D = 128
E = 320000

NC = 2
NS = 16
NW = NC * NS
EPW = E // NW
K = 80
NCHUNK = EPW // K
G = 5
NGI = NCHUNK // G
P = 2
Q = 3
NP = 10240
RPT = NP // NS


def _sc_agg_body(feat_hbm, ei_hbm, zf_hbm, zd_hbm,
                 psum_hbm, pdeg_hbm,
                 colbuf, rowbuf, fb, degbuf,
                 accum, semg, semsc, semic, semir):
    row_hbm = ei_hbm.at[0]
    col_hbm = ei_hbm.at[1]
    c = lax.axis_index("c")
    s = lax.axis_index("s")
    wid = s * NC + c

    pltpu.sync_copy(zf_hbm, accum.at[pl.ds(s * RPT, RPT)])
    pltpu.sync_copy(zd_hbm, degbuf)
    plsc.subcore_barrier()

    pltpu.sync_copy(col_hbm.at[wid, 0], colbuf.at[pl.ds(0, G)])
    pltpu.sync_copy(row_hbm.at[wid, 0], rowbuf.at[pl.ds(0, G)])
    pltpu.async_copy(col_hbm.at[wid, 1], colbuf.at[pl.ds(G, G)], semic)
    pltpu.async_copy(row_hbm.at[wid, 1], rowbuf.at[pl.ds(G, G)], semir)

    ones16 = jnp.ones((16,), jnp.float32)

    for b in range(P):
        pltpu.async_copy(feat_hbm.at[colbuf.at[b]],
                         fb.at[pl.ds(b * K, K)], semg.at[b])

    def chunk(ch, carry):
        grp = ch // G
        j = ch - grp * G
        slot = lax.rem(grp, 2)
        sg = lax.rem(ch, Q)

        @pl.when(jnp.logical_and(j == G - P, grp + 1 < NGI))
        def _():
            pltpu.make_async_copy(col_hbm.at[wid, grp + 1],
                                  colbuf.at[pl.ds((1 - slot) * G, G)],
                                  semic).wait()
            pltpu.make_async_copy(row_hbm.at[wid, grp + 1],
                                  rowbuf.at[pl.ds((1 - slot) * G, G)],
                                  semir).wait()

        rw = slot * G + j
        pltpu.make_async_copy(feat_hbm.at[colbuf.at[rw]],
                              fb.at[pl.ds(sg * K, K)], semg.at[sg]).wait()
        pltpu.async_copy(fb.at[pl.ds(sg * K, K)],
                         accum.at[rowbuf.at[rw]], semsc.at[sg], add=True)
        for t in range(K // 16):
            idx = rowbuf[rw, pl.ds(t * 16, 16)]
            plsc.addupdate_scatter(degbuf, [idx], ones16)

        nxt = ch + P
        sn_fb = lax.rem(nxt, Q)

        @pl.when(jnp.logical_and(nxt < NCHUNK, ch >= Q - P))
        def _():
            pltpu.make_async_copy(fb.at[pl.ds(sn_fb * K, K)],
                                  accum.at[rowbuf.at[rw]],
                                  semsc.at[sn_fb]).wait()

        @pl.when(nxt < NCHUNK)
        def _():
            gn = nxt // G
            jn = nxt - gn * G
            sn = lax.rem(gn, 2)
            pltpu.async_copy(feat_hbm.at[colbuf.at[sn * G + jn]],
                             fb.at[pl.ds(sn_fb * K, K)], semg.at[sn_fb])

        @pl.when(jnp.logical_and(j == G - 1, grp + 2 < NGI))
        def _():
            pltpu.async_copy(col_hbm.at[wid, grp + 2],
                             colbuf.at[pl.ds(slot * G, G)], semic)
            pltpu.async_copy(row_hbm.at[wid, grp + 2],
                             rowbuf.at[pl.ds(slot * G, G)], semir)

        return carry

    lax.fori_loop(0, NCHUNK, chunk, 0)

    for q in range(Q):
        pltpu.make_async_copy(fb.at[pl.ds(q * K, K)],
                              accum.at[rowbuf.at[0]], semsc.at[q]).wait()
    plsc.subcore_barrier()

    out_base = c * NP + s * RPT
    pltpu.sync_copy(accum.at[pl.ds(s * RPT, RPT)],
                    psum_hbm.at[pl.ds(out_base, RPT)])
    pltpu.sync_copy(degbuf, pdeg_hbm.at[wid])


_sc_agg = functools.partial(
    pl.kernel,
    out_type=(
        jax.ShapeDtypeStruct((2 * NP, D), jnp.float32),
        jax.ShapeDtypeStruct((NW, NP), jnp.float32),
    ),
    mesh=plsc.VectorSubcoreMesh(core_axis_name="c", subcore_axis_name="s",
                                num_cores=NC, num_subcores=NS),
    compiler_params=pltpu.CompilerParams(needs_layout_passes=False),
    scratch_types=[
        pltpu.VMEM((2 * G, K), jnp.int32),
        pltpu.VMEM((2 * G, K), jnp.int32),
        pltpu.VMEM((Q * K, D), jnp.float32),
        pltpu.VMEM((NP,), jnp.float32),
        pltpu.VMEM_SHARED((NP, D), jnp.float32),
        pltpu.SemaphoreType.DMA((Q,)),
        pltpu.SemaphoreType.DMA((Q,)),
        pltpu.SemaphoreType.DMA,
        pltpu.SemaphoreType.DMA,
    ],
)(_sc_agg_body)


def _ln(h, sc, of):
    m = jnp.mean(h, axis=1, keepdims=True)
    v = jnp.mean((h - m) ** 2, axis=1, keepdims=True) + 1e-9
    return (h - m) * sc * lax.rsqrt(v) + of


_DN = (((1,), (1,)), ((), ()))


def _tc_self_body(feat_ref, ws_ref, bs_ref, scs_ref, ofs_ref, out_ref):
    hs = lax.dot_general(feat_ref[...], ws_ref[...], _DN,
                         preferred_element_type=jnp.float32)
    hs = jnp.maximum(hs + bs_ref[...], 0.0)
    out_ref[...] = _ln(hs, scs_ref[...], ofs_ref[...])


def _tc_neigh_body(ps_ref, pd_ref, hs_ref, wn_ref, bn_ref, scn_ref, ofn_ref,
                   out_ref, dg_ref):
    i = pl.program_id(0)

    @pl.when(i == 0)
    def _():
        dn0 = (((0,), (0,)), ((), ()))
        dg = lax.dot_general(pd_ref[...], jnp.ones((NW, 1), jnp.float32),
                             dn0, preferred_element_type=jnp.float32)
        dg_ref[...] = jnp.maximum(dg, 1.0)

    ns = ps_ref[0] + ps_ref[1]
    fn = ns / dg_ref[pl.ds(i * BR, BR), :]
    hn = lax.dot_general(fn, wn_ref[...], _DN,
                         preferred_element_type=jnp.float32)
    hn = jnp.maximum(hn + bn_ref[...], 0.0)
    out_ref[...] = hs_ref[...] + _ln(hn, scn_ref[...], ofn_ref[...])


BR = 400


def _tc_self(feat, W_self, b_self, sc_s, of_s):
    full = lambda shape: pl.BlockSpec(shape, lambda i: (0,) * len(shape))
    return pl.pallas_call(
        _tc_self_body,
        grid=(N // BR,),
        in_specs=[
            pl.BlockSpec((BR, D), lambda i: (i, 0)),
            full((D, D)),
            full((1, D)),
            full((1, D)),
            full((1, D)),
        ],
        out_specs=pl.BlockSpec((BR, D), lambda i: (i, 0)),
        out_shape=jax.ShapeDtypeStruct((N, D), jnp.float32),
    )(feat, W_self, b_self, sc_s, of_s)


def _tc_neigh(psum, pdeg, hself, W_neigh, b_neigh, sc_n, of_n):
    full = lambda shape: pl.BlockSpec(shape, lambda i: (0,) * len(shape))
    return pl.pallas_call(
        _tc_neigh_body,
        grid=(N // BR,),
        in_specs=[
            pl.BlockSpec((2, BR, D), lambda i: (0, i, 0)),
            full((NW, NP)),
            pl.BlockSpec((BR, D), lambda i: (i, 0)),
            full((D, D)),
            full((1, D)),
            full((1, D)),
            full((1, D)),
        ],
        out_specs=pl.BlockSpec((BR, D), lambda i: (i, 0)),
        out_shape=jax.ShapeDtypeStruct((N, D), jnp.float32),
        scratch_shapes=[pltpu.VMEM((NP, 1), jnp.float32)],
    )(psum, pdeg, hself, W_neigh, b_neigh, sc_n, of_n)


def kernel(feat_in, edge_index, W_self, b_self, W_neigh, b_neigh, offset, scale):
    ei = edge_index.reshape(2, NW, NGI, G, K)
    zf = jnp.zeros((RPT, D), jnp.float32)
    zd = jnp.zeros((NP,), jnp.float32)

    hself = _tc_self(feat_in, W_self, b_self.reshape(1, D),
                     scale[:D].reshape(1, D), offset[:D].reshape(1, D))
    psum, pdeg = _sc_agg(feat_in, ei, zf, zd)

    return _tc_neigh(
        psum.reshape(2, NP, D), pdeg, hself,
        W_neigh, b_neigh.reshape(1, D),
        scale[D:].reshape(1, D), offset[D:].reshape(1, D),
    )
